# Initial kernel scaffold; baseline (speedup 1.0000x reference)
import jax
import jax.numpy as jnp
from jax.experimental import pallas as pl


def kernel(edge_index, W, b):
    raise NotImplementedError("write your pallas kernel here")


# speedup vs baseline: 34.9465x; 34.9465x over previous
import functools

import jax
import jax.numpy as jnp
from jax import lax
from jax.experimental import pallas as pl
from jax.experimental.pallas import tpu as pltpu
from jax.experimental.pallas import tpu_sc as plsc

N_NODES_P = 10240
N_EDGES_P = 327680
CHUNK = 128
CHUNKS_PER_TILE = 80
PAD_BIN = 10016

_MESH = plsc.VectorSubcoreMesh(core_axis_name="c", subcore_axis_name="s")


@functools.partial(
    pl.kernel,
    mesh=_MESH,
    out_type=jax.ShapeDtypeStruct((2, N_NODES_P), jnp.float32),
    scratch_types=[
        pltpu.VMEM((CHUNKS_PER_TILE, CHUNK), jnp.int32),
        pltpu.VMEM((CHUNK,), jnp.float32),
        pltpu.VMEM((N_NODES_P // 16,), jnp.float32),
        pltpu.VMEM_SHARED((N_NODES_P,), jnp.float32),
    ],
)
def _sc_degree_hist(dst_hbm, out_hbm, idx_v, ones_v, zero_v, hist_sh):
    c = lax.axis_index("c")
    s = lax.axis_index("s")
    pltpu.sync_copy(dst_hbm.at[c, s], idx_v)
    for i in range(CHUNK // 16):
        ones_v[pl.ds(i * 16, 16)] = jnp.ones((16,), jnp.float32)
    for i in range(N_NODES_P // 16 // 16):
        zero_v[pl.ds(i * 16, 16)] = jnp.zeros((16,), jnp.float32)
    sl = N_NODES_P // 16
    pltpu.sync_copy(zero_v, hist_sh.at[pl.ds(s * sl, sl)])
    plsc.subcore_barrier()

    def body(j, carry):
        pltpu.sync_copy(ones_v, hist_sh.at[idx_v.at[j]], add=True)
        return carry

    lax.fori_loop(0, CHUNKS_PER_TILE, body, 0)
    plsc.subcore_barrier()
    pltpu.sync_copy(hist_sh.at[pl.ds(s * sl, sl)], out_hbm.at[c, pl.ds(s * sl, sl)])


def _tc_linear_body(part_ref, w_ref, b_ref, out_ref):
    s_row = jnp.sum(w_ref[...], axis=1)
    s_rep = jnp.broadcast_to(s_row[None, :], (2, 128))
    acc = lax.dot_general(
        part_ref[...], s_rep, (((0,), (0,)), ((), ())),
        preferred_element_type=jnp.float32,
    )
    out_ref[...] = acc + b_ref[...]


def kernel(edge_index, W, b):
    dst = edge_index[1]
    dst_p = jnp.pad(dst, (0, N_EDGES_P - dst.shape[0]), constant_values=PAD_BIN)
    dst_r = dst_p.reshape(2, 16, CHUNKS_PER_TILE, CHUNK)
    part = _sc_degree_hist(dst_r)

    blk = 1280
    grid = N_NODES_P // blk
    out_full = pl.pallas_call(
        _tc_linear_body,
        grid=(grid,),
        in_specs=[
            pl.BlockSpec((2, blk), lambda i: (0, i)),
            pl.BlockSpec((128, 128), lambda i: (0, 0)),
            pl.BlockSpec((1, 128), lambda i: (0, 0)),
        ],
        out_specs=pl.BlockSpec((blk, 128), lambda i: (i, 0)),
        out_shape=jax.ShapeDtypeStruct((N_NODES_P, 128), jnp.float32),
    )(part, W, b.reshape(1, 128))
    return out_full[:10000]


# speedup vs baseline: 36.9046x; 1.0560x over previous
import functools

import jax
import jax.numpy as jnp
from jax import lax
from jax.experimental import pallas as pl
from jax.experimental.pallas import tpu as pltpu
from jax.experimental.pallas import tpu_sc as plsc

N_NODES = 10000
N_NODES_P = 10240
N_EDGES_P = 327680
PER_TILE = N_EDGES_P // 32
PAD_BIN = 10016

_MESH = plsc.VectorSubcoreMesh(core_axis_name="c", subcore_axis_name="s")


@functools.partial(
    pl.kernel,
    mesh=_MESH,
    out_type=jax.ShapeDtypeStruct((2, N_NODES_P), jnp.float32),
    scratch_types=[
        pltpu.VMEM((PER_TILE,), jnp.int32),
        pltpu.VMEM((PER_TILE,), jnp.float32),
        pltpu.VMEM((N_NODES_P // 16,), jnp.float32),
        pltpu.VMEM_SHARED((N_NODES_P,), jnp.float32),
    ],
)
def _sc_degree_hist(dst_hbm, out_hbm, idx_v, ones_v, zero_v, hist_sh):
    c = lax.axis_index("c")
    s = lax.axis_index("s")
    pltpu.sync_copy(dst_hbm.at[c, s], idx_v)

    def fill(j, carry):
        ones_v[pl.ds(j * 16, 16)] = jnp.ones((16,), jnp.float32)
        return carry

    lax.fori_loop(0, PER_TILE // 16, fill, 0)
    for i in range(N_NODES_P // 16 // 16):
        zero_v[pl.ds(i * 16, 16)] = jnp.zeros((16,), jnp.float32)
    sl = N_NODES_P // 16
    pltpu.sync_copy(zero_v, hist_sh.at[pl.ds(s * sl, sl)])
    plsc.subcore_barrier()

    pltpu.sync_copy(ones_v, hist_sh.at[idx_v], add=True)
    plsc.subcore_barrier()
    pltpu.sync_copy(hist_sh.at[pl.ds(s * sl, sl)], out_hbm.at[c, pl.ds(s * sl, sl)])


def _tc_linear_body(part_ref, w_ref, b_ref, out_ref):
    deg2 = part_ref[:, :N_NODES]
    s_row = jnp.sum(w_ref[...], axis=1)
    s_rep = jnp.broadcast_to(s_row[None, :], (2, 128))
    acc = lax.dot_general(
        deg2, s_rep, (((0,), (0,)), ((), ())),
        preferred_element_type=jnp.float32,
    )
    out_ref[...] = acc + b_ref[...]


def kernel(edge_index, W, b):
    dst = edge_index[1]
    dst_p = jnp.pad(dst, (0, N_EDGES_P - dst.shape[0]), constant_values=PAD_BIN)
    dst_r = dst_p.reshape(2, 16, PER_TILE)
    part = _sc_degree_hist(dst_r)

    out = pl.pallas_call(
        _tc_linear_body,
        out_shape=jax.ShapeDtypeStruct((N_NODES, 128), jnp.float32),
    )(part, W, b.reshape(1, 128))
    return out


# speedup vs baseline: 55.2529x; 1.4972x over previous
import functools

import jax
import jax.numpy as jnp
from jax import lax
from jax.experimental import pallas as pl
from jax.experimental.pallas import tpu as pltpu
from jax.experimental.pallas import tpu_sc as plsc

N_NODES = 10000
N_NODES_P = 10240
N_EDGES = 320000
PER_TILE = N_EDGES // 32

_MESH = plsc.VectorSubcoreMesh(core_axis_name="c", subcore_axis_name="s")


@functools.partial(
    pl.kernel,
    mesh=_MESH,
    out_type=jax.ShapeDtypeStruct((2, N_NODES_P), jnp.float32),
    scratch_types=[
        pltpu.VMEM((PER_TILE,), jnp.int32),
        pltpu.VMEM((PER_TILE,), jnp.float32),
        pltpu.VMEM((N_NODES_P // 16,), jnp.float32),
        pltpu.VMEM_SHARED((N_NODES_P,), jnp.float32),
    ],
)
def _sc_degree_hist(edges_hbm, out_hbm, idx_v, ones_v, zero_v, hist_sh):
    c = lax.axis_index("c")
    s = lax.axis_index("s")
    w = c * 16 + s
    pltpu.sync_copy(edges_hbm.at[1, w], idx_v)

    def fill(j, carry):
        ones_v[pl.ds(j * 16, 16)] = jnp.ones((16,), jnp.float32)
        return carry

    lax.fori_loop(0, PER_TILE // 16, fill, 0)
    for i in range(N_NODES_P // 16 // 16):
        zero_v[pl.ds(i * 16, 16)] = jnp.zeros((16,), jnp.float32)
    sl = N_NODES_P // 16
    pltpu.sync_copy(zero_v, hist_sh.at[pl.ds(s * sl, sl)])
    plsc.subcore_barrier()

    pltpu.sync_copy(ones_v, hist_sh.at[idx_v], add=True)
    plsc.subcore_barrier()
    pltpu.sync_copy(hist_sh.at[pl.ds(s * sl, sl)], out_hbm.at[c, pl.ds(s * sl, sl)])


def _tc_linear_body(part_ref, w_ref, b_ref, out_ref):
    deg2 = part_ref[:, :N_NODES]
    s_row = jnp.sum(w_ref[...], axis=1)
    s_rep = jnp.broadcast_to(s_row[None, :], (2, 128))
    acc = lax.dot_general(
        deg2, s_rep, (((0,), (0,)), ((), ())),
        preferred_element_type=jnp.float32,
    )
    out_ref[...] = acc + b_ref[...]


def kernel(edge_index, W, b):
    edges_r = edge_index.reshape(2, 32, PER_TILE)
    part = _sc_degree_hist(edges_r)

    out = pl.pallas_call(
        _tc_linear_body,
        out_shape=jax.ShapeDtypeStruct((N_NODES, 128), jnp.float32),
    )(part, W, b.reshape(1, 128))
    return out
